# Initial kernel scaffold; baseline (speedup 1.0000x reference)
import functools

import jax
import jax.numpy as jnp
from jax import lax
from jax.experimental import pallas as pl
from jax.experimental.pallas import tpu as pltpu
from jax.experimental.pallas import tpu_sc as plsc

N = 10000
E = 320000
D = 128
H = 128
G = 64

NC = 2
NS = 16
NW = NC * NS
LANES = 16

N_PAD = 10240
ROWS_PER_TILE_E = 80
E_PAD = NW * ROWS_PER_TILE_E * 128
E_ROWS = E_PAD // 128
NODES_PER_TILE = N_PAD // NW
ZCHUNK = 128
K_IDX = 4

_mesh = plsc.VectorSubcoreMesh(core_axis_name="c", subcore_axis_name="s")


@functools.partial(
    pl.kernel,
    out_type=jax.ShapeDtypeStruct((NC, N_PAD, D), jnp.float32),
    mesh=_mesh,
    scratch_types=[
        pltpu.VMEM_SHARED((N_PAD, D), jnp.float32),
        pltpu.VMEM((K_IDX, 128), jnp.int32),
        pltpu.VMEM((K_IDX, 128), jnp.int32),
        pltpu.VMEM((K_IDX, 128), jnp.float32),
        pltpu.VMEM((128, D), jnp.float32),
        pltpu.VMEM((ZCHUNK, D), jnp.float32),
    ],
)
def _sc_aggregate(x_hbm, src_hbm, dst_hbm, w_hbm, out_hbm,
                  accum, srcb, dstb, wb, rows, zbuf):
    c = lax.axis_index("c")
    s = lax.axis_index("s")
    tid = c * NS + s

    zv = jnp.zeros((LANES,), jnp.float32)

    @pl.loop(0, ZCHUNK)
    def _(rr):
        for k in range(D // LANES):
            zbuf[rr, pl.ds(k * LANES, LANES)] = zv

    rows_per_sub = N_PAD // NS

    @pl.loop(0, rows_per_sub // ZCHUNK)
    def _(i):
        pltpu.sync_copy(zbuf, accum.at[pl.ds(s * rows_per_sub + i * ZCHUNK, ZCHUNK)])

    plsc.subcore_barrier()

    base = tid * ROWS_PER_TILE_E

    @pl.loop(0, ROWS_PER_TILE_E, step=K_IDX)
    def _(g):
        pltpu.sync_copy(src_hbm.at[pl.ds(base + g, K_IDX)], srcb)
        pltpu.sync_copy(dst_hbm.at[pl.ds(base + g, K_IDX)], dstb)
        pltpu.sync_copy(w_hbm.at[pl.ds(base + g, K_IDX)], wb)
        for j in range(K_IDX):
            pltpu.sync_copy(x_hbm.at[srcb.at[j]], rows)

            @pl.loop(0, 128)
            def _(rr):
                wv = jnp.full((LANES,), wb[j, rr], jnp.float32)
                for k in range(D // LANES):
                    sl = pl.ds(k * LANES, LANES)
                    rows[rr, sl] = rows[rr, sl] * wv

            pltpu.sync_copy(rows, accum.at[dstb.at[j]], add=True)

    plsc.subcore_barrier()

    @pl.loop(0, rows_per_sub // ZCHUNK)
    def _(i):
        off = s * rows_per_sub + i * ZCHUNK
        pltpu.sync_copy(accum.at[pl.ds(off, ZCHUNK)],
                        out_hbm.at[c, pl.ds(off, ZCHUNK)])


BN = 1024


def _tc_qr_body(parts, xr, w1rel, b1, w1root, w2rel, w2root, wlin, b2, qr):
    agg = parts[0] + parts[1]
    dn = (((1,), (1,)), ((), ()))
    h = lax.dot_general(agg, w1rel[...], dn, preferred_element_type=jnp.float32)
    hr = lax.dot_general(xr[...], w1root[...], dn, preferred_element_type=jnp.float32)
    h1 = jnp.maximum(h + hr + b1[...], 0.0)
    dk = (((1,), (0,)), ((), ()))
    vrel = lax.dot_general(wlin[...], w2rel[...], dk, preferred_element_type=jnp.float32)
    vroot = lax.dot_general(wlin[...], w2root[...], dk, preferred_element_type=jnp.float32)
    q = lax.dot_general(vrel, h1, dn, preferred_element_type=jnp.float32)
    r = lax.dot_general(vroot, h1, dn, preferred_element_type=jnp.float32)
    c2 = jnp.sum(wlin[...] * b2[...])
    qr[...] = jnp.concatenate([q, r + c2], axis=0)


_tc_qr = pl.pallas_call(
    _tc_qr_body,
    grid=(N_PAD // BN,),
    in_specs=[
        pl.BlockSpec((NC, BN, D), lambda i: (0, i, 0)),
        pl.BlockSpec((BN, D), lambda i: (i, 0)),
        pl.BlockSpec((H, D), lambda i: (0, 0)),
        pl.BlockSpec((1, H), lambda i: (0, 0)),
        pl.BlockSpec((H, D), lambda i: (0, 0)),
        pl.BlockSpec((H, H), lambda i: (0, 0)),
        pl.BlockSpec((H, H), lambda i: (0, 0)),
        pl.BlockSpec((1, H), lambda i: (0, 0)),
        pl.BlockSpec((1, H), lambda i: (0, 0)),
    ],
    out_specs=pl.BlockSpec((2, BN), lambda i: (0, i)),
    out_shape=jax.ShapeDtypeStruct((2, N_PAD), jnp.float32),
)


@functools.partial(
    pl.kernel,
    out_type=jax.ShapeDtypeStruct((NW, 192), jnp.float32),
    mesh=_mesh,
    scratch_types=[
        pltpu.VMEM((N_PAD,), jnp.float32),
        pltpu.VMEM((N_PAD,), jnp.float32),
        pltpu.VMEM((N_PAD,), jnp.int32),
        pltpu.VMEM((ROWS_PER_TILE_E, 128), jnp.int32),
        pltpu.VMEM((ROWS_PER_TILE_E, 128), jnp.int32),
        pltpu.VMEM((ROWS_PER_TILE_E, 128), jnp.float32),
        pltpu.VMEM((LANES * G,), jnp.float32),
        pltpu.VMEM((LANES * G,), jnp.float32),
        pltpu.VMEM((LANES * G,), jnp.float32),
        pltpu.VMEM((192,), jnp.float32),
    ],
)
def _sc_reduce(qr_hbm, batch_hbm, src_hbm, dst_hbm, w_hbm, out_hbm,
               qv, rv, bv, srcb, dstb, wb, zacc, sacc, cacc, partial):
    c = lax.axis_index("c")
    s = lax.axis_index("s")
    tid = c * NS + s

    pltpu.sync_copy(qr_hbm.at[0], qv)
    pltpu.sync_copy(qr_hbm.at[1], rv)
    pltpu.sync_copy(batch_hbm, bv)
    pltpu.sync_copy(src_hbm.at[pl.ds(tid * ROWS_PER_TILE_E, ROWS_PER_TILE_E)], srcb)
    pltpu.sync_copy(dst_hbm.at[pl.ds(tid * ROWS_PER_TILE_E, ROWS_PER_TILE_E)], dstb)
    pltpu.sync_copy(w_hbm.at[pl.ds(tid * ROWS_PER_TILE_E, ROWS_PER_TILE_E)], wb)

    zv = jnp.zeros((LANES,), jnp.float32)

    @pl.loop(0, LANES * G, step=LANES)
    def _(i):
        zacc[pl.ds(i, LANES)] = zv
        sacc[pl.ds(i, LANES)] = zv
        cacc[pl.ds(i, LANES)] = zv

    lane = lax.iota(jnp.int32, LANES)
    loff = lane * G

    @pl.loop(0, ROWS_PER_TILE_E)
    def _(rr):
        for k in range(128 // LANES):
            sl = pl.ds(k * LANES, LANES)
            sv = srcb[rr, sl]
            dv = dstb[rr, sl]
            wv = wb[rr, sl]
            qg = plsc.load_gather(qv, [sv])
            bg = plsc.load_gather(bv, [dv])
            plsc.addupdate_scatter(zacc, [loff + bg], wv * qg)

    base_n = tid * NODES_PER_TILE
    ones = jnp.ones((LANES,), jnp.float32)

    @pl.loop(0, NODES_PER_TILE, step=LANES)
    def _(i):
        idx = base_n + i + lane
        valid = idx < N
        rv16 = rv[pl.ds(base_n + i, LANES)]
        bv16 = bv[pl.ds(base_n + i, LANES)]
        plsc.addupdate_scatter(sacc, [loff + bv16], rv16, mask=valid)
        plsc.addupdate_scatter(cacc, [loff + bv16], ones, mask=valid)

    for cg in range(G // LANES):
        az = jnp.zeros((LANES,), jnp.float32)
        asq = jnp.zeros((LANES,), jnp.float32)
        ac = jnp.zeros((LANES,), jnp.float32)
        for row in range(LANES):
            off = pl.ds(row * G + cg * LANES, LANES)
            az = az + zacc[off]
            asq = asq + sacc[off]
            ac = ac + cacc[off]
        partial[pl.ds(cg * LANES, LANES)] = az
        partial[pl.ds(G + cg * LANES, LANES)] = asq
        partial[pl.ds(2 * G + cg * LANES, LANES)] = ac

    pltpu.sync_copy(partial, out_hbm.at[tid])


def _tc_final_body(p, bl, out):
    t = jnp.sum(p[...], axis=0)
    z = t[0:G]
    sv = t[G:2 * G]
    cnt = t[2 * G:3 * G]
    out[...] = ((z + sv) / jnp.maximum(cnt, 1.0) + bl[0, 0])[:, None]


_tc_final = pl.pallas_call(
    _tc_final_body,
    in_specs=[
        pl.BlockSpec((NW, 192), lambda: (0, 0)),
        pl.BlockSpec((1, 1), lambda: (0, 0)),
    ],
    out_specs=pl.BlockSpec((G, 1), lambda: (0, 0)),
    out_shape=jax.ShapeDtypeStruct((G, 1), jnp.float32),
)


def kernel(x, edge_index, edge_weight, batch, W1_rel, b1_rel, W1_root,
           W2_rel, b2_rel, W2_root, W_lin, b_lin):
    x_pad = jnp.pad(x, ((0, N_PAD - N), (0, 0)))
    batch_pad = jnp.pad(batch, (0, N_PAD - N))
    epad = E_PAD - E
    src = jnp.pad(edge_index[0], (0, epad)).reshape(E_ROWS, 128)
    dst = jnp.pad(edge_index[1], (0, epad)).reshape(E_ROWS, 128)
    w = jnp.pad(edge_weight, (0, epad)).reshape(E_ROWS, 128)

    parts = _sc_aggregate(x_pad, src, dst, w)
    qr = _tc_qr(parts, x_pad, W1_rel, b1_rel.reshape(1, H), W1_root,
                W2_rel, W2_root, W_lin, b2_rel.reshape(1, H))
    tile_partials = _sc_reduce(qr, batch_pad, src, dst, w)
    out = _tc_final(tile_partials, b_lin.reshape(1, 1))
    return out


# speedup vs baseline: 5.0631x; 5.0631x over previous
import dataclasses
import functools

import jax
import jax.numpy as jnp
from jax import lax
from jax.experimental import pallas as pl
from jax.experimental.pallas import tpu as pltpu
from jax.experimental.pallas import tpu_sc as plsc

N = 10000
E = 320000
D = 128
H = 128
G = 64

NC = 2
NS = 16
NW = NC * NS
LANES = 16

N_PAD = 10240
ROWS_PER_TILE_E = 80
E_PAD = NW * ROWS_PER_TILE_E * 128
E_ROWS = E_PAD // 128
NODES_PER_TILE = N_PAD // NW
ZCHUNK = 128
K_IDX = 4

_mesh = plsc.VectorSubcoreMesh(core_axis_name="c", subcore_axis_name="s")

_sc_params = pltpu.CompilerParams()
if "needs_layout_passes" in pltpu.CompilerParams.__dataclass_fields__:
    _sc_params = dataclasses.replace(_sc_params, needs_layout_passes=False)


@functools.partial(
    pl.kernel,
    out_type=jax.ShapeDtypeStruct((NC, N_PAD, D), jnp.float32),
    mesh=_mesh,
    compiler_params=_sc_params,
    scratch_types=[
        pltpu.VMEM_SHARED((N_PAD, D), jnp.float32),
        pltpu.VMEM((K_IDX, 128), jnp.int32),
        pltpu.VMEM((K_IDX, 128), jnp.int32),
        pltpu.VMEM((K_IDX, 128), jnp.float32),
        pltpu.VMEM((128, D), jnp.float32),
        pltpu.VMEM((ZCHUNK, D), jnp.float32),
    ],
)
def _sc_aggregate(x_hbm, src_hbm, dst_hbm, w_hbm, out_hbm,
                  accum, srcb, dstb, wb, rows, zbuf):
    c = lax.axis_index("c")
    s = lax.axis_index("s")
    tid = c * NS + s

    zv = jnp.zeros((LANES,), jnp.float32)

    @pl.loop(0, ZCHUNK)
    def _(rr):
        for k in range(D // LANES):
            zbuf[rr, pl.ds(k * LANES, LANES)] = zv

    rows_per_sub = N_PAD // NS

    @pl.loop(0, rows_per_sub // ZCHUNK)
    def _(i):
        pltpu.sync_copy(zbuf, accum.at[pl.ds(s * rows_per_sub + i * ZCHUNK, ZCHUNK)])

    plsc.subcore_barrier()

    base = tid * ROWS_PER_TILE_E

    @pl.loop(0, ROWS_PER_TILE_E, step=K_IDX)
    def _(g):
        pltpu.sync_copy(src_hbm.at[pl.ds(base + g, K_IDX)], srcb)
        pltpu.sync_copy(dst_hbm.at[pl.ds(base + g, K_IDX)], dstb)
        pltpu.sync_copy(w_hbm.at[pl.ds(base + g, K_IDX)], wb)
        for j in range(K_IDX):
            pltpu.sync_copy(x_hbm.at[srcb.at[j]], rows)

            @pl.loop(0, 128)
            def _(rr):
                jv = jnp.full((LANES,), j, jnp.int32)
                rv_idx = jnp.full((LANES,), rr, jnp.int32)
                wv = plsc.load_gather(wb, [jv, rv_idx])
                for k in range(D // LANES):
                    sl = pl.ds(k * LANES, LANES)
                    rows[rr, sl] = rows[rr, sl] * wv

            pltpu.sync_copy(rows, accum.at[dstb.at[j]], add=True)

    plsc.subcore_barrier()

    @pl.loop(0, rows_per_sub // ZCHUNK)
    def _(i):
        off = s * rows_per_sub + i * ZCHUNK
        pltpu.sync_copy(accum.at[pl.ds(off, ZCHUNK)],
                        out_hbm.at[c, pl.ds(off, ZCHUNK)])


BN = 1024


def _tc_qr_body(parts, xr, w1rel, b1, w1root, w2rel, w2root, wlin, b2, qr):
    agg = parts[0] + parts[1]
    dn = (((1,), (1,)), ((), ()))
    dot = functools.partial(lax.dot_general,
                            precision=lax.Precision.HIGHEST,
                            preferred_element_type=jnp.float32)
    h = dot(agg, w1rel[...], dn)
    hr = dot(xr[...], w1root[...], dn)
    h1 = jnp.maximum(h + hr + b1[...], 0.0)
    dk = (((1,), (0,)), ((), ()))
    vrel = dot(wlin[...], w2rel[...], dk)
    vroot = dot(wlin[...], w2root[...], dk)
    q = dot(vrel, h1, dn)
    r = dot(vroot, h1, dn)
    c2 = jnp.sum(wlin[...] * b2[...])
    qr[...] = jnp.concatenate([q, r + c2], axis=0)


_tc_qr = pl.pallas_call(
    _tc_qr_body,
    grid=(N_PAD // BN,),
    in_specs=[
        pl.BlockSpec((NC, BN, D), lambda i: (0, i, 0)),
        pl.BlockSpec((BN, D), lambda i: (i, 0)),
        pl.BlockSpec((H, D), lambda i: (0, 0)),
        pl.BlockSpec((1, H), lambda i: (0, 0)),
        pl.BlockSpec((H, D), lambda i: (0, 0)),
        pl.BlockSpec((H, H), lambda i: (0, 0)),
        pl.BlockSpec((H, H), lambda i: (0, 0)),
        pl.BlockSpec((1, H), lambda i: (0, 0)),
        pl.BlockSpec((1, H), lambda i: (0, 0)),
    ],
    out_specs=pl.BlockSpec((2, BN), lambda i: (0, i)),
    out_shape=jax.ShapeDtypeStruct((2, N_PAD), jnp.float32),
)


@functools.partial(
    pl.kernel,
    out_type=jax.ShapeDtypeStruct((NW, 192), jnp.float32),
    mesh=_mesh,
    compiler_params=_sc_params,
    scratch_types=[
        pltpu.VMEM((N_PAD,), jnp.float32),
        pltpu.VMEM((N_PAD,), jnp.float32),
        pltpu.VMEM((N_PAD,), jnp.int32),
        pltpu.VMEM((ROWS_PER_TILE_E, 128), jnp.int32),
        pltpu.VMEM((ROWS_PER_TILE_E, 128), jnp.int32),
        pltpu.VMEM((ROWS_PER_TILE_E, 128), jnp.float32),
        pltpu.VMEM((LANES * G,), jnp.float32),
        pltpu.VMEM((LANES * G,), jnp.float32),
        pltpu.VMEM((LANES * G,), jnp.float32),
        pltpu.VMEM((192,), jnp.float32),
    ],
)
def _sc_reduce(qr_hbm, batch_hbm, src_hbm, dst_hbm, w_hbm, out_hbm,
               qv, rv, bv, srcb, dstb, wb, zacc, sacc, cacc, partial):
    c = lax.axis_index("c")
    s = lax.axis_index("s")
    tid = c * NS + s

    pltpu.sync_copy(qr_hbm.at[0], qv)
    pltpu.sync_copy(qr_hbm.at[1], rv)
    pltpu.sync_copy(batch_hbm, bv)
    pltpu.sync_copy(src_hbm.at[pl.ds(tid * ROWS_PER_TILE_E, ROWS_PER_TILE_E)], srcb)
    pltpu.sync_copy(dst_hbm.at[pl.ds(tid * ROWS_PER_TILE_E, ROWS_PER_TILE_E)], dstb)
    pltpu.sync_copy(w_hbm.at[pl.ds(tid * ROWS_PER_TILE_E, ROWS_PER_TILE_E)], wb)

    zv = jnp.zeros((LANES,), jnp.float32)

    @pl.loop(0, LANES * G, step=LANES)
    def _(i):
        zacc[pl.ds(i, LANES)] = zv
        sacc[pl.ds(i, LANES)] = zv
        cacc[pl.ds(i, LANES)] = zv

    lane = lax.iota(jnp.int32, LANES)
    loff = lane * G

    @pl.loop(0, ROWS_PER_TILE_E)
    def _(rr):
        for k in range(128 // LANES):
            sl = pl.ds(k * LANES, LANES)
            sv = srcb[rr, sl]
            dv = dstb[rr, sl]
            wv = wb[rr, sl]
            qg = plsc.load_gather(qv, [sv])
            bg = plsc.load_gather(bv, [dv])
            plsc.addupdate_scatter(zacc, [loff + bg], wv * qg)

    base_n = tid * NODES_PER_TILE
    ones = jnp.ones((LANES,), jnp.float32)

    @pl.loop(0, NODES_PER_TILE, step=LANES)
    def _(i):
        idx = base_n + i + lane
        valid = idx < N
        rv16 = rv[pl.ds(base_n + i, LANES)]
        bv16 = bv[pl.ds(base_n + i, LANES)]
        plsc.addupdate_scatter(sacc, [loff + bv16], rv16, mask=valid)
        plsc.addupdate_scatter(cacc, [loff + bv16], ones, mask=valid)

    for cg in range(G // LANES):
        az = jnp.zeros((LANES,), jnp.float32)
        asq = jnp.zeros((LANES,), jnp.float32)
        ac = jnp.zeros((LANES,), jnp.float32)
        for row in range(LANES):
            off = pl.ds(row * G + cg * LANES, LANES)
            az = az + zacc[off]
            asq = asq + sacc[off]
            ac = ac + cacc[off]
        partial[pl.ds(cg * LANES, LANES)] = az
        partial[pl.ds(G + cg * LANES, LANES)] = asq
        partial[pl.ds(2 * G + cg * LANES, LANES)] = ac

    pltpu.sync_copy(partial, out_hbm.at[tid])


def _tc_final_body(p, bl, out):
    t = jnp.sum(p[...], axis=0)
    z = t[0:G]
    sv = t[G:2 * G]
    cnt = t[2 * G:3 * G]
    out[...] = ((z + sv) / jnp.maximum(cnt, 1.0) + bl[0, 0])[:, None]


_tc_final = pl.pallas_call(
    _tc_final_body,
    in_specs=[
        pl.BlockSpec((NW, 192), lambda: (0, 0)),
        pl.BlockSpec((1, 1), lambda: (0, 0)),
    ],
    out_specs=pl.BlockSpec((G, 1), lambda: (0, 0)),
    out_shape=jax.ShapeDtypeStruct((G, 1), jnp.float32),
)


def kernel(x, edge_index, edge_weight, batch, W1_rel, b1_rel, W1_root,
           W2_rel, b2_rel, W2_root, W_lin, b_lin):
    x_pad = jnp.pad(x, ((0, N_PAD - N), (0, 0)))
    batch_pad = jnp.pad(batch, (0, N_PAD - N))
    epad = E_PAD - E
    src = jnp.pad(edge_index[0], (0, epad)).reshape(E_ROWS, 128)
    dst = jnp.pad(edge_index[1], (0, epad)).reshape(E_ROWS, 128)
    w = jnp.pad(edge_weight, (0, epad)).reshape(E_ROWS, 128)

    parts = _sc_aggregate(x_pad, src, dst, w)
    qr = _tc_qr(parts, x_pad, W1_rel, b1_rel.reshape(1, H), W1_root,
                W2_rel, W2_root, W_lin, b2_rel.reshape(1, H))
    tile_partials = _sc_reduce(qr, batch_pad, src, dst, w)
    out = _tc_final(tile_partials, b_lin.reshape(1, 1))
    return out


# speedup vs baseline: 6.4037x; 1.2648x over previous
import dataclasses
import functools

import jax
import jax.numpy as jnp
from jax import lax
from jax.experimental import pallas as pl
from jax.experimental.pallas import tpu as pltpu
from jax.experimental.pallas import tpu_sc as plsc

N = 10000
E = 320000
D = 128
H = 128
G = 64

NC = 2
NS = 16
NW = NC * NS
LANES = 16

N_PAD = 10240
ROWS_PER_TILE_E = 80
E_PAD = NW * ROWS_PER_TILE_E * 128
E_ROWS = E_PAD // 128
NODES_PER_TILE = N_PAD // NW
ZCHUNK = 128
K_IDX = 4

_mesh = plsc.VectorSubcoreMesh(core_axis_name="c", subcore_axis_name="s")

_sc_params = pltpu.CompilerParams()
if "needs_layout_passes" in pltpu.CompilerParams.__dataclass_fields__:
    _sc_params = dataclasses.replace(_sc_params, needs_layout_passes=False)


EGRP = 16


@functools.partial(
    pl.kernel,
    out_type=jax.ShapeDtypeStruct((NC, N_PAD, D), jnp.float32),
    mesh=_mesh,
    compiler_params=_sc_params,
    scratch_types=[
        pltpu.VMEM_SHARED((N_PAD, D), jnp.float32),
        pltpu.VMEM((EGRP, 128), jnp.int32),
        pltpu.VMEM((EGRP, 128), jnp.int32),
        pltpu.VMEM((EGRP, 128), jnp.float32),
        pltpu.VMEM((128, D), jnp.float32),
        pltpu.VMEM((128, D), jnp.float32),
        pltpu.SemaphoreType.DMA((2,)),
        pltpu.SemaphoreType.DMA((2,)),
    ],
)
def _sc_aggregate(x_hbm, src_hbm, dst_hbm, w_hbm, out_hbm, accum, srcb, dstb, wb, r0, r1, gsem, ssem):
    c = lax.axis_index("c")
    s = lax.axis_index("s")
    tid = c * NS + s
    rows = [r0, r1]

    zv = jnp.zeros((LANES,), jnp.float32)

    @pl.loop(0, ZCHUNK)
    def _(rr):
        for k in range(D // LANES):
            r0[rr, pl.ds(k * LANES, LANES)] = zv

    rows_per_sub = N_PAD // NS

    @pl.loop(0, rows_per_sub // ZCHUNK)
    def _(i):
        pltpu.sync_copy(r0, accum.at[pl.ds(s * rows_per_sub + i * ZCHUNK, ZCHUNK)])

    plsc.subcore_barrier()

    base = tid * ROWS_PER_TILE_E

    def stage(grp):
        pltpu.sync_copy(src_hbm.at[pl.ds(base + grp * EGRP, EGRP)], srcb)
        pltpu.sync_copy(dst_hbm.at[pl.ds(base + grp * EGRP, EGRP)], dstb)
        pltpu.sync_copy(w_hbm.at[pl.ds(base + grp * EGRP, EGRP)], wb)

    def start_gather(jj, b):
        pltpu.async_copy(x_hbm.at[srcb.at[jj]], rows[b], gsem.at[b])

    def wait_gather(jj, b):
        pltpu.make_async_copy(x_hbm.at[srcb.at[jj]], rows[b],
                              gsem.at[b]).wait()

    def start_scatter(jj, b):
        pltpu.async_copy(rows[b], accum.at[dstb.at[jj]], ssem.at[b],
                         add=True)

    def wait_scatter(jj, b):
        pltpu.make_async_copy(rows[b], accum.at[dstb.at[jj]],
                              ssem.at[b]).wait()

    def scale(jj, b):
        buf = rows[b]

        @pl.loop(0, 128, step=2)
        def _(rr):
            for u in range(2):
                ri = rr + u
                wv = plsc.load_gather(wb, [jnp.full((LANES,), jj, jnp.int32),
                                           jnp.full((LANES,), ri, jnp.int32)])
                for k in range(D // LANES):
                    sl = pl.ds(k * LANES, LANES)
                    buf[ri, sl] = buf[ri, sl] * wv

    NGRP = ROWS_PER_TILE_E // EGRP
    stage(0)
    start_gather(0, 0)

    @pl.loop(0, NGRP)
    def _(grp):
        for jj in range(EGRP):
            b = jj % 2
            if jj >= 1:
                wait_scatter(jj - 1, 1 - b)
            if jj + 1 < EGRP:
                start_gather(jj + 1, 1 - b)
            wait_gather(jj, b)
            scale(jj, b)
            start_scatter(jj, b)
        wait_scatter(EGRP - 1, 1)

        @pl.when(grp + 1 < NGRP)
        def _():
            stage(grp + 1)
            start_gather(0, 0)

    plsc.subcore_barrier()

    @pl.loop(0, rows_per_sub // ZCHUNK)
    def _(i):
        off = s * rows_per_sub + i * ZCHUNK
        pltpu.sync_copy(accum.at[pl.ds(off, ZCHUNK)],
                        out_hbm.at[c, pl.ds(off, ZCHUNK)])


BN = 1024


def _tc_qr_body(parts, xr, w1rel, b1, w1root, w2rel, w2root, wlin, b2, qr):
    agg = parts[0] + parts[1]
    dn = (((1,), (1,)), ((), ()))
    dot = functools.partial(lax.dot_general,
                            precision=lax.Precision.HIGHEST,
                            preferred_element_type=jnp.float32)
    h = dot(agg, w1rel[...], dn)
    hr = dot(xr[...], w1root[...], dn)
    h1 = jnp.maximum(h + hr + b1[...], 0.0)
    dk = (((1,), (0,)), ((), ()))
    vrel = dot(wlin[...], w2rel[...], dk)
    vroot = dot(wlin[...], w2root[...], dk)
    q = dot(vrel, h1, dn)
    r = dot(vroot, h1, dn)
    c2 = jnp.sum(wlin[...] * b2[...])
    qr[...] = jnp.concatenate([q, r + c2], axis=0)


_tc_qr = pl.pallas_call(
    _tc_qr_body,
    grid=(N_PAD // BN,),
    in_specs=[
        pl.BlockSpec((NC, BN, D), lambda i: (0, i, 0)),
        pl.BlockSpec((BN, D), lambda i: (i, 0)),
        pl.BlockSpec((H, D), lambda i: (0, 0)),
        pl.BlockSpec((1, H), lambda i: (0, 0)),
        pl.BlockSpec((H, D), lambda i: (0, 0)),
        pl.BlockSpec((H, H), lambda i: (0, 0)),
        pl.BlockSpec((H, H), lambda i: (0, 0)),
        pl.BlockSpec((1, H), lambda i: (0, 0)),
        pl.BlockSpec((1, H), lambda i: (0, 0)),
    ],
    out_specs=pl.BlockSpec((2, BN), lambda i: (0, i)),
    out_shape=jax.ShapeDtypeStruct((2, N_PAD), jnp.float32),
)


@functools.partial(
    pl.kernel,
    out_type=jax.ShapeDtypeStruct((NW, 192), jnp.float32),
    mesh=_mesh,
    compiler_params=_sc_params,
    scratch_types=[
        pltpu.VMEM((N_PAD,), jnp.float32),
        pltpu.VMEM((N_PAD,), jnp.float32),
        pltpu.VMEM((N_PAD,), jnp.int32),
        pltpu.VMEM((ROWS_PER_TILE_E, 128), jnp.int32),
        pltpu.VMEM((ROWS_PER_TILE_E, 128), jnp.int32),
        pltpu.VMEM((ROWS_PER_TILE_E, 128), jnp.float32),
        pltpu.VMEM((LANES * G,), jnp.float32),
        pltpu.VMEM((LANES * G,), jnp.float32),
        pltpu.VMEM((LANES * G,), jnp.float32),
        pltpu.VMEM((192,), jnp.float32),
    ],
)
def _sc_reduce(qr_hbm, batch_hbm, src_hbm, dst_hbm, w_hbm, out_hbm,
               qv, rv, bv, srcb, dstb, wb, zacc, sacc, cacc, partial):
    c = lax.axis_index("c")
    s = lax.axis_index("s")
    tid = c * NS + s

    pltpu.sync_copy(qr_hbm.at[0], qv)
    pltpu.sync_copy(qr_hbm.at[1], rv)
    pltpu.sync_copy(batch_hbm, bv)
    pltpu.sync_copy(src_hbm.at[pl.ds(tid * ROWS_PER_TILE_E, ROWS_PER_TILE_E)], srcb)
    pltpu.sync_copy(dst_hbm.at[pl.ds(tid * ROWS_PER_TILE_E, ROWS_PER_TILE_E)], dstb)
    pltpu.sync_copy(w_hbm.at[pl.ds(tid * ROWS_PER_TILE_E, ROWS_PER_TILE_E)], wb)

    zv = jnp.zeros((LANES,), jnp.float32)

    @pl.loop(0, LANES * G, step=LANES)
    def _(i):
        zacc[pl.ds(i, LANES)] = zv
        sacc[pl.ds(i, LANES)] = zv
        cacc[pl.ds(i, LANES)] = zv

    lane = lax.iota(jnp.int32, LANES)
    loff = lane * G

    @pl.loop(0, ROWS_PER_TILE_E)
    def _(rr):
        for k in range(128 // LANES):
            sl = pl.ds(k * LANES, LANES)
            sv = srcb[rr, sl]
            dv = dstb[rr, sl]
            wv = wb[rr, sl]
            qg = plsc.load_gather(qv, [sv])
            bg = plsc.load_gather(bv, [dv])
            plsc.addupdate_scatter(zacc, [loff + bg], wv * qg)

    base_n = tid * NODES_PER_TILE
    ones = jnp.ones((LANES,), jnp.float32)

    @pl.loop(0, NODES_PER_TILE, step=LANES)
    def _(i):
        idx = base_n + i + lane
        valid = idx < N
        rv16 = rv[pl.ds(base_n + i, LANES)]
        bv16 = bv[pl.ds(base_n + i, LANES)]
        plsc.addupdate_scatter(sacc, [loff + bv16], rv16, mask=valid)
        plsc.addupdate_scatter(cacc, [loff + bv16], ones, mask=valid)

    for cg in range(G // LANES):
        az = jnp.zeros((LANES,), jnp.float32)
        asq = jnp.zeros((LANES,), jnp.float32)
        ac = jnp.zeros((LANES,), jnp.float32)
        for row in range(LANES):
            off = pl.ds(row * G + cg * LANES, LANES)
            az = az + zacc[off]
            asq = asq + sacc[off]
            ac = ac + cacc[off]
        partial[pl.ds(cg * LANES, LANES)] = az
        partial[pl.ds(G + cg * LANES, LANES)] = asq
        partial[pl.ds(2 * G + cg * LANES, LANES)] = ac

    pltpu.sync_copy(partial, out_hbm.at[tid])


def _tc_final_body(p, bl, out):
    t = jnp.sum(p[...], axis=0)
    z = t[0:G]
    sv = t[G:2 * G]
    cnt = t[2 * G:3 * G]
    out[...] = ((z + sv) / jnp.maximum(cnt, 1.0) + bl[0, 0])[:, None]


_tc_final = pl.pallas_call(
    _tc_final_body,
    in_specs=[
        pl.BlockSpec((NW, 192), lambda: (0, 0)),
        pl.BlockSpec((1, 1), lambda: (0, 0)),
    ],
    out_specs=pl.BlockSpec((G, 1), lambda: (0, 0)),
    out_shape=jax.ShapeDtypeStruct((G, 1), jnp.float32),
)


def kernel(x, edge_index, edge_weight, batch, W1_rel, b1_rel, W1_root,
           W2_rel, b2_rel, W2_root, W_lin, b_lin):
    x_pad = jnp.pad(x, ((0, N_PAD - N), (0, 0)))
    batch_pad = jnp.pad(batch, (0, N_PAD - N))
    epad = E_PAD - E
    src = jnp.pad(edge_index[0], (0, epad)).reshape(E_ROWS, 128)
    dst = jnp.pad(edge_index[1], (0, epad)).reshape(E_ROWS, 128)
    w = jnp.pad(edge_weight, (0, epad)).reshape(E_ROWS, 128)
    parts = _sc_aggregate(x_pad, src, dst, w)
    qr = _tc_qr(parts, x_pad, W1_rel, b1_rel.reshape(1, H), W1_root,
                W2_rel, W2_root, W_lin, b2_rel.reshape(1, H))
    tile_partials = _sc_reduce(qr, batch_pad, src, dst, w)
    out = _tc_final(tile_partials, b_lin.reshape(1, 1))
    return out


# speedup vs baseline: 7.7744x; 1.2141x over previous
import dataclasses
import functools

import jax
import jax.numpy as jnp
from jax import lax
from jax.experimental import pallas as pl
from jax.experimental.pallas import tpu as pltpu
from jax.experimental.pallas import tpu_sc as plsc

N = 10000
E = 320000
D = 128
H = 128
G = 64

NC = 2
NS = 16
NW = NC * NS
LANES = 16

N_PAD = 10240
ROWS_PER_TILE_E = 80
E_PAD = NW * ROWS_PER_TILE_E * 128
E_ROWS = E_PAD // 128
NODES_PER_TILE = N_PAD // NW
ZCHUNK = 128
K_IDX = 4

_mesh = plsc.VectorSubcoreMesh(core_axis_name="c", subcore_axis_name="s")

_sc_params = pltpu.CompilerParams()
if "needs_layout_passes" in pltpu.CompilerParams.__dataclass_fields__:
    _sc_params = dataclasses.replace(_sc_params, needs_layout_passes=False)
_sc_params_lin = _sc_params
if "use_tc_tiling_on_sc" in pltpu.CompilerParams.__dataclass_fields__:
    _sc_params_lin = dataclasses.replace(_sc_params, use_tc_tiling_on_sc=False)


EGRP = 16


@functools.partial(
    pl.kernel,
    out_type=jax.ShapeDtypeStruct((NC, N_PAD, D), jnp.float32),
    mesh=_mesh,
    compiler_params=_sc_params_lin,
    scratch_types=[
        pltpu.VMEM_SHARED((N_PAD, D), jnp.float32),
        pltpu.VMEM((EGRP, 128), jnp.int32),
        pltpu.VMEM((EGRP, 128), jnp.int32),
        pltpu.VMEM((EGRP, 128), jnp.float32),
        pltpu.VMEM((128, D // 2), jnp.int32),
        pltpu.VMEM((128, D // 2), jnp.int32),
        pltpu.VMEM((128, D), jnp.float32),
        pltpu.SemaphoreType.DMA((2,)),
        pltpu.SemaphoreType.DMA((2,)),
    ],
)
def _sc_aggregate(x_hbm, src_hbm, dst_hbm, w_hbm, out_hbm, accum,
                  srcb, dstb, wb, rb0, rb1, fbuf, gsem, ssem):
    c = lax.axis_index("c")
    s = lax.axis_index("s")
    tid = c * NS + s
    rows = [rb0, rb1]

    zv = jnp.zeros((LANES,), jnp.float32)

    @pl.loop(0, ZCHUNK)
    def _(rr):
        for k in range(D // LANES):
            fbuf[rr, pl.ds(k * LANES, LANES)] = zv

    rows_per_sub = N_PAD // NS

    @pl.loop(0, rows_per_sub // ZCHUNK)
    def _(i):
        pltpu.sync_copy(fbuf, accum.at[pl.ds(s * rows_per_sub + i * ZCHUNK, ZCHUNK)])

    plsc.subcore_barrier()

    base = tid * ROWS_PER_TILE_E

    def stage(grp):
        pltpu.sync_copy(src_hbm.at[pl.ds(base + grp * EGRP, EGRP)], srcb)
        pltpu.sync_copy(dst_hbm.at[pl.ds(base + grp * EGRP, EGRP)], dstb)
        pltpu.sync_copy(w_hbm.at[pl.ds(base + grp * EGRP, EGRP)], wb)

    def start_gather(jj, b):
        pltpu.async_copy(x_hbm.at[srcb.at[jj]], rows[b], gsem.at[b])

    def wait_gather(jj, b):
        pltpu.make_async_copy(x_hbm.at[srcb.at[jj]], rows[b],
                              gsem.at[b]).wait()

    def start_scatter(jj):
        pltpu.async_copy(fbuf, accum.at[dstb.at[jj]], ssem.at[0],
                         add=True)

    def wait_scatter(jj):
        pltpu.make_async_copy(fbuf, accum.at[dstb.at[jj]],
                              ssem.at[0]).wait()

    def scale(jj, b):
        buf = rows[b]

        @pl.loop(0, 128)
        def _(ri):
            wv = plsc.load_gather(wb, [jnp.full((LANES,), jj, jnp.int32),
                                       jnp.full((LANES,), ri, jnp.int32)])
            for g2 in range(D // 32):
                packed = buf[ri, pl.ds(g2 * LANES, LANES)]
                ab = plsc.bitcast(packed, jnp.bfloat16)
                lo, hi = plsc.unpack(ab, format=plsc.PackFormat.INTERLEAVED)
                fbuf[ri, pl.ds(g2 * 32, LANES)] = lo * wv
                fbuf[ri, pl.ds(g2 * 32 + LANES, LANES)] = hi * wv

    NGRP = ROWS_PER_TILE_E // EGRP
    stage(0)
    start_gather(0, 0)

    @pl.loop(0, NGRP)
    def _(grp):
        for jj in range(EGRP):
            b = jj % 2
            if jj + 1 < EGRP:
                start_gather(jj + 1, 1 - b)
            if jj >= 1:
                wait_scatter(jj - 1)
            wait_gather(jj, b)
            scale(jj, b)
            start_scatter(jj)
        wait_scatter(EGRP - 1)

        @pl.when(grp + 1 < NGRP)
        def _():
            stage(grp + 1)
            start_gather(0, 0)

    plsc.subcore_barrier()

    @pl.loop(0, rows_per_sub // ZCHUNK)
    def _(i):
        off = s * rows_per_sub + i * ZCHUNK
        pltpu.sync_copy(accum.at[pl.ds(off, ZCHUNK)],
                        out_hbm.at[c, pl.ds(off, ZCHUNK)])


BN = 1024


def _tc_qr_body(parts, xr, w1rel, b1, w1root, w2rel, w2root, wlin, b2, qr):
    agg = parts[0] + parts[1]
    dn = (((1,), (1,)), ((), ()))
    dot = functools.partial(lax.dot_general,
                            precision=lax.Precision.HIGHEST,
                            preferred_element_type=jnp.float32)
    h = dot(agg, w1rel[...], dn)
    hr = dot(xr[...], w1root[...], dn)
    h1 = jnp.maximum(h + hr + b1[...], 0.0)
    dk = (((1,), (0,)), ((), ()))
    vrel = dot(wlin[...], w2rel[...], dk)
    vroot = dot(wlin[...], w2root[...], dk)
    q = dot(vrel, h1, dn)
    r = dot(vroot, h1, dn)
    c2 = jnp.sum(wlin[...] * b2[...])
    qr[...] = jnp.concatenate([q, r + c2], axis=0)


_tc_qr = pl.pallas_call(
    _tc_qr_body,
    grid=(N_PAD // BN,),
    in_specs=[
        pl.BlockSpec((NC, BN, D), lambda i: (0, i, 0)),
        pl.BlockSpec((BN, D), lambda i: (i, 0)),
        pl.BlockSpec((H, D), lambda i: (0, 0)),
        pl.BlockSpec((1, H), lambda i: (0, 0)),
        pl.BlockSpec((H, D), lambda i: (0, 0)),
        pl.BlockSpec((H, H), lambda i: (0, 0)),
        pl.BlockSpec((H, H), lambda i: (0, 0)),
        pl.BlockSpec((1, H), lambda i: (0, 0)),
        pl.BlockSpec((1, H), lambda i: (0, 0)),
    ],
    out_specs=pl.BlockSpec((2, BN), lambda i: (0, i)),
    out_shape=jax.ShapeDtypeStruct((2, N_PAD), jnp.float32),
)


@functools.partial(
    pl.kernel,
    out_type=jax.ShapeDtypeStruct((NW, 192), jnp.float32),
    mesh=_mesh,
    compiler_params=_sc_params,
    scratch_types=[
        pltpu.VMEM((N_PAD,), jnp.float32),
        pltpu.VMEM((N_PAD,), jnp.float32),
        pltpu.VMEM((N_PAD,), jnp.int32),
        pltpu.VMEM((ROWS_PER_TILE_E, 128), jnp.int32),
        pltpu.VMEM((ROWS_PER_TILE_E, 128), jnp.int32),
        pltpu.VMEM((ROWS_PER_TILE_E, 128), jnp.float32),
        pltpu.VMEM((LANES * G,), jnp.float32),
        pltpu.VMEM((LANES * G,), jnp.float32),
        pltpu.VMEM((LANES * G,), jnp.float32),
        pltpu.VMEM((192,), jnp.float32),
    ],
)
def _sc_reduce(qr_hbm, batch_hbm, src_hbm, dst_hbm, w_hbm, out_hbm,
               qv, rv, bv, srcb, dstb, wb, zacc, sacc, cacc, partial):
    c = lax.axis_index("c")
    s = lax.axis_index("s")
    tid = c * NS + s

    pltpu.sync_copy(qr_hbm.at[0], qv)
    pltpu.sync_copy(qr_hbm.at[1], rv)
    pltpu.sync_copy(batch_hbm, bv)
    pltpu.sync_copy(src_hbm.at[pl.ds(tid * ROWS_PER_TILE_E, ROWS_PER_TILE_E)], srcb)
    pltpu.sync_copy(dst_hbm.at[pl.ds(tid * ROWS_PER_TILE_E, ROWS_PER_TILE_E)], dstb)
    pltpu.sync_copy(w_hbm.at[pl.ds(tid * ROWS_PER_TILE_E, ROWS_PER_TILE_E)], wb)

    zv = jnp.zeros((LANES,), jnp.float32)

    @pl.loop(0, LANES * G, step=LANES)
    def _(i):
        zacc[pl.ds(i, LANES)] = zv
        sacc[pl.ds(i, LANES)] = zv
        cacc[pl.ds(i, LANES)] = zv

    lane = lax.iota(jnp.int32, LANES)
    loff = lane * G

    @pl.loop(0, ROWS_PER_TILE_E)
    def _(rr):
        for k in range(128 // LANES):
            sl = pl.ds(k * LANES, LANES)
            sv = srcb[rr, sl]
            dv = dstb[rr, sl]
            wv = wb[rr, sl]
            qg = plsc.load_gather(qv, [sv])
            bg = plsc.load_gather(bv, [dv])
            plsc.addupdate_scatter(zacc, [loff + bg], wv * qg)

    base_n = tid * NODES_PER_TILE
    ones = jnp.ones((LANES,), jnp.float32)

    @pl.loop(0, NODES_PER_TILE, step=LANES)
    def _(i):
        idx = base_n + i + lane
        valid = idx < N
        rv16 = rv[pl.ds(base_n + i, LANES)]
        bv16 = bv[pl.ds(base_n + i, LANES)]
        plsc.addupdate_scatter(sacc, [loff + bv16], rv16, mask=valid)
        plsc.addupdate_scatter(cacc, [loff + bv16], ones, mask=valid)

    for cg in range(G // LANES):
        az = jnp.zeros((LANES,), jnp.float32)
        asq = jnp.zeros((LANES,), jnp.float32)
        ac = jnp.zeros((LANES,), jnp.float32)
        for row in range(LANES):
            off = pl.ds(row * G + cg * LANES, LANES)
            az = az + zacc[off]
            asq = asq + sacc[off]
            ac = ac + cacc[off]
        partial[pl.ds(cg * LANES, LANES)] = az
        partial[pl.ds(G + cg * LANES, LANES)] = asq
        partial[pl.ds(2 * G + cg * LANES, LANES)] = ac

    pltpu.sync_copy(partial, out_hbm.at[tid])


def _tc_final_body(p, bl, out):
    t = jnp.sum(p[...], axis=0)
    z = t[0:G]
    sv = t[G:2 * G]
    cnt = t[2 * G:3 * G]
    out[...] = ((z + sv) / jnp.maximum(cnt, 1.0) + bl[0, 0])[:, None]


_tc_final = pl.pallas_call(
    _tc_final_body,
    in_specs=[
        pl.BlockSpec((NW, 192), lambda: (0, 0)),
        pl.BlockSpec((1, 1), lambda: (0, 0)),
    ],
    out_specs=pl.BlockSpec((G, 1), lambda: (0, 0)),
    out_shape=jax.ShapeDtypeStruct((G, 1), jnp.float32),
)


def kernel(x, edge_index, edge_weight, batch, W1_rel, b1_rel, W1_root,
           W2_rel, b2_rel, W2_root, W_lin, b_lin):
    x_pad = jnp.pad(x, ((0, N_PAD - N), (0, 0)))
    batch_pad = jnp.pad(batch, (0, N_PAD - N))
    perm = []
    for g2 in range(D // 32):
        for i in range(16):
            perm.extend([g2 * 32 + i, g2 * 32 + 16 + i])
    x_bf = x_pad[:, jnp.array(perm, dtype=jnp.int32)].astype(jnp.bfloat16)
    x_bfp = lax.bitcast_convert_type(x_bf.reshape(N_PAD, D // 2, 2),
                                     jnp.int32)
    epad = E_PAD - E
    src = jnp.pad(edge_index[0], (0, epad)).reshape(E_ROWS, 128)
    dst = jnp.pad(edge_index[1], (0, epad)).reshape(E_ROWS, 128)
    w = jnp.pad(edge_weight, (0, epad)).reshape(E_ROWS, 128)
    parts = _sc_aggregate(x_bfp, src, dst, w)
    qr = _tc_qr(parts, x_pad, W1_rel, b1_rel.reshape(1, H), W1_root,
                W2_rel, W2_root, W_lin, b2_rel.reshape(1, H))
    tile_partials = _sc_reduce(qr, batch_pad, src, dst, w)
    out = _tc_final(tile_partials, b_lin.reshape(1, 1))
    return out
